# baseline (speedup 1.0000x reference)
import jax
import jax.numpy as jnp
from jax import lax
from jax.experimental import pallas as pl
from jax.experimental.pallas import tpu as pltpu
from jax.experimental.pallas import tpu_sc as plsc

_NUM_CORES = 2
_NUM_SUBCORES = 16
_NW = _NUM_CORES * _NUM_SUBCORES
_LANES = 16
_CHUNK = 64
_NBUF = 12
_PRE = 6


def _make_gather(N, R, K, D):
    b_per_w = K * N // _NW
    n_vec = b_per_w // _LANES
    sizes = []
    rem = b_per_w
    while rem:
        c = min(_CHUNK, rem)
        sizes.append(c)
        rem -= c
    offs = [sum(sizes[:i]) for i in range(len(sizes))]
    n_chunks = len(sizes)

    mesh = plsc.VectorSubcoreMesh(
        core_axis_name="c",
        subcore_axis_name="s",
        num_cores=_NUM_CORES,
        num_subcores=_NUM_SUBCORES,
    )

    def body(table_hbm, idx_hbm, out_hbm, idx_v, gidx_v, bufs, sems):
        wid = lax.axis_index("s") * _NUM_CORES + lax.axis_index("c")
        base = wid * b_per_w

        pltpu.sync_copy(idx_hbm.at[pl.ds(base, b_per_w)], idx_v)

        lanes = lax.iota(jnp.int32, _LANES)
        c_n = jnp.full((_LANES,), N, jnp.int32)
        c_r = jnp.full((_LANES,), R, jnp.int32)
        for j in range(n_vec):
            p = lanes + jnp.full((_LANES,), base + j * _LANES, jnp.int32)
            n = p - lax.div(p, c_n) * c_n
            gidx_v[pl.ds(j * _LANES, _LANES)] = (
                idx_v[pl.ds(j * _LANES, _LANES)] + n * c_r
            )

        gathers = [None] * n_chunks
        writes = [None] * n_chunks

        def start_gather(c):
            s = c % _NBUF
            gathers[c] = pltpu.async_copy(
                table_hbm.at[gidx_v.at[pl.ds(offs[c], sizes[c])]],
                bufs[s].at[pl.ds(0, sizes[c])],
                sems[s],
            )

        for b in range(min(_PRE, n_chunks)):
            start_gather(b)
        waited = set()
        for c in range(n_chunks):
            nxt = c + _PRE
            if nxt < n_chunks:
                if nxt >= _NBUF:
                    writes[nxt - _NBUF].wait()
                    waited.add(nxt - _NBUF)
                start_gather(nxt)
            gathers[c].wait()
            writes[c] = pltpu.async_copy(
                bufs[c % _NBUF].at[pl.ds(0, sizes[c])],
                out_hbm.at[pl.ds(base + offs[c], sizes[c])],
                sems[_NBUF + c % _NBUF],
            )
        for c in range(n_chunks):
            if c not in waited:
                writes[c].wait()

    return pl.kernel(
        body,
        out_type=jax.ShapeDtypeStruct((K * N, D), jnp.float32),
        mesh=mesh,
        scratch_types=[
            pltpu.VMEM((b_per_w,), jnp.int32),
            pltpu.VMEM((b_per_w,), jnp.int32),
            [pltpu.VMEM((_CHUNK, D), jnp.float32) for _ in range(_NBUF)],
            [pltpu.SemaphoreType.DMA for _ in range(2 * _NBUF)],
        ],
    )


def kernel(input_tensor, indices):
    N, R, D = input_tensor.shape
    K = indices.shape[1]
    assert (K * N) % _NW == 0 and (K * N // _NW) % _LANES == 0

    table = input_tensor.reshape(N * R, D)
    idx_t = indices.T.reshape(K * N).astype(jnp.int32)
    out = _make_gather(N, R, K, D)(table, idx_t)
    return out.reshape(K, N, D).transpose(1, 0, 2)


# speedup vs baseline: 1.0206x; 1.0206x over previous
import jax
import jax.numpy as jnp
from jax import lax
from jax.experimental import pallas as pl
from jax.experimental.pallas import tpu as pltpu
from jax.experimental.pallas import tpu_sc as plsc

_NUM_CORES = 2
_NUM_SUBCORES = 16
_NW = _NUM_CORES * _NUM_SUBCORES
_LANES = 16
_CHUNK = 128
_NBUF = 7
_PRE = 4


def _make_gather(N, R, K, D):
    b_per_w = K * N // _NW
    n_vec = b_per_w // _LANES
    sizes = []
    rem = b_per_w
    while rem:
        c = min(_CHUNK, rem)
        sizes.append(c)
        rem -= c
    offs = [sum(sizes[:i]) for i in range(len(sizes))]
    n_chunks = len(sizes)

    mesh = plsc.VectorSubcoreMesh(
        core_axis_name="c",
        subcore_axis_name="s",
        num_cores=_NUM_CORES,
        num_subcores=_NUM_SUBCORES,
    )

    def body(table_hbm, idx_hbm, out_hbm, idx_v, gidx_v, bufs, sems):
        wid = lax.axis_index("s") * _NUM_CORES + lax.axis_index("c")
        base = wid * b_per_w

        pltpu.sync_copy(idx_hbm.at[pl.ds(base, b_per_w)], idx_v)

        lanes = lax.iota(jnp.int32, _LANES)
        c_n = jnp.full((_LANES,), N, jnp.int32)
        c_r = jnp.full((_LANES,), R, jnp.int32)
        for j in range(n_vec):
            p = lanes + jnp.full((_LANES,), base + j * _LANES, jnp.int32)
            n = p - lax.div(p, c_n) * c_n
            gidx_v[pl.ds(j * _LANES, _LANES)] = (
                idx_v[pl.ds(j * _LANES, _LANES)] + n * c_r
            )

        gathers = [None] * n_chunks
        writes = [None] * n_chunks

        def start_gather(c):
            s = c % _NBUF
            gathers[c] = pltpu.async_copy(
                table_hbm.at[gidx_v.at[pl.ds(offs[c], sizes[c])]],
                bufs[s].at[pl.ds(0, sizes[c])],
                sems[s],
            )

        for b in range(min(_PRE, n_chunks)):
            start_gather(b)
        waited = set()
        for c in range(n_chunks):
            nxt = c + _PRE
            if nxt < n_chunks:
                if nxt >= _NBUF:
                    writes[nxt - _NBUF].wait()
                    waited.add(nxt - _NBUF)
                start_gather(nxt)
            gathers[c].wait()
            writes[c] = pltpu.async_copy(
                bufs[c % _NBUF].at[pl.ds(0, sizes[c])],
                out_hbm.at[pl.ds(base + offs[c], sizes[c])],
                sems[_NBUF + c % _NBUF],
            )
        for c in range(n_chunks):
            if c not in waited:
                writes[c].wait()

    return pl.kernel(
        body,
        out_type=jax.ShapeDtypeStruct((K * N, D), jnp.float32),
        mesh=mesh,
        scratch_types=[
            pltpu.VMEM((b_per_w,), jnp.int32),
            pltpu.VMEM((b_per_w,), jnp.int32),
            [pltpu.VMEM((_CHUNK, D), jnp.float32) for _ in range(_NBUF)],
            [pltpu.SemaphoreType.DMA for _ in range(2 * _NBUF)],
        ],
    )


def kernel(input_tensor, indices):
    N, R, D = input_tensor.shape
    K = indices.shape[1]
    assert (K * N) % _NW == 0 and (K * N // _NW) % _LANES == 0

    table = input_tensor.reshape(N * R, D)
    idx_t = indices.T.reshape(K * N).astype(jnp.int32)
    out = _make_gather(N, R, K, D)(table, idx_t)
    return out.reshape(K, N, D).transpose(1, 0, 2)
